# baseline (speedup 1.0000x reference)
import functools

import jax
import jax.numpy as jnp
from jax import lax
from jax.experimental import pallas as pl
from jax.experimental.pallas import tpu as pltpu
from jax.experimental.pallas import tpu_sc as plsc

BATCH = 16384
VOCABP1 = 100001
DIM = 64
FIELDS = 4
LANES = 16

_NC = 2
_NS = 16
_NW = _NC * _NS
_R = BATCH // _NW
_CG = DIM // LANES

_TBLK = 512
_TGRID = (VOCABP1 + _TBLK - 1) // _TBLK

_mesh = plsc.VectorSubcoreMesh(core_axis_name="c", subcore_axis_name="s")
_params = pltpu.CompilerParams(use_tc_tiling_on_sc=False)


_LINROWS = 256 * _TGRID


def _transpose_body(tt_ref, out_ref):
    x = tt_ref[...]
    eye = jnp.eye(DIM, dtype=jnp.float32)
    y = jax.lax.dot_general(x, eye, (((0,), (0,)), ((), ())),
                            preferred_element_type=jnp.float32)
    out_ref[...] = jnp.concatenate([y[:256, :], y[256:, :]], axis=1)


_transpose_flat = pl.pallas_call(
    _transpose_body,
    grid=(_TGRID,),
    in_specs=[pl.BlockSpec((DIM, _TBLK), lambda j: (0, j))],
    out_specs=pl.BlockSpec((_TBLK // 2, 128), lambda j: (j, 0)),
    out_shape=jax.ShapeDtypeStruct((_LINROWS, 128), jnp.float32),
)


def _permute_indices(idx_v):
    def ibody(i, carry):
        v = idx_v[pl.ds(i * LANES, LANES)]
        t = v & 511
        two_t = t + t
        idx_v[pl.ds(i * LANES, LANES)] = (v - t) + jnp.where(
            t < 256, two_t, two_t - 511)
        return carry

    lax.fori_loop(0, _R // LANES, ibody, 0)


@functools.partial(
    pl.kernel,
    out_type=jax.ShapeDtypeStruct((BATCH, DIM), jnp.float32),
    mesh=_mesh,
    scratch_types=[
        pltpu.VMEM((_R,), jnp.int32),
        pltpu.VMEM((_R, DIM), jnp.float32),
        pltpu.SemaphoreType.DMA,
    ],
    compiler_params=_params,
)
def _gather_first(lab, tab, out, idx_v, gbuf, sem):
    wid = lax.axis_index("s") * _NC + lax.axis_index("c")
    base = wid * _R
    pltpu.sync_copy(lab.at[pl.ds(base, _R)], idx_v)
    _permute_indices(idx_v)
    pltpu.async_copy(tab.at[idx_v], gbuf, sem).wait()
    pltpu.sync_copy(gbuf, out.at[pl.ds(base, _R)])


@functools.partial(
    pl.kernel,
    out_type=jax.ShapeDtypeStruct((BATCH, DIM), jnp.float32),
    mesh=_mesh,
    scratch_types=[
        pltpu.VMEM((_R,), jnp.int32),
        pltpu.VMEM((_R, DIM), jnp.float32),
        pltpu.VMEM((_R, DIM), jnp.float32),
        pltpu.SemaphoreType.DMA,
        pltpu.SemaphoreType.DMA,
    ],
    compiler_params=_params,
)
def _gather_acc(lab, tab, acc, out, idx_v, gbuf, abuf, gsem, asem):
    wid = lax.axis_index("s") * _NC + lax.axis_index("c")
    base = wid * _R
    pltpu.sync_copy(lab.at[pl.ds(base, _R)], idx_v)
    _permute_indices(idx_v)
    gd = pltpu.async_copy(tab.at[idx_v], gbuf, gsem)
    ad = pltpu.async_copy(acc.at[pl.ds(base, _R)], abuf, asem)
    gd.wait()
    ad.wait()

    def body(r, carry):
        for cg in range(_CG):
            sl = pl.ds(cg * LANES, LANES)
            gbuf[r, sl] = gbuf[r, sl] + abuf[r, sl]
        return carry

    lax.fori_loop(0, _R, body, 0)
    pltpu.sync_copy(gbuf, out.at[pl.ds(base, _R)])


def kernel(labels_f0, labels_f1, labels_f2, labels_f3,
           table_f0, table_f1, table_f2, table_f3):
    labels = [labels_f0, labels_f1, labels_f2, labels_f3]
    tables = [table_f0, table_f1, table_f2, table_f3]
    lins = [_transpose_flat(t.T).reshape(2 * _LINROWS, DIM) for t in tables]
    acc = _gather_first(labels[0], lins[0])
    for f in range(1, FIELDS):
        acc = _gather_acc(labels[f], lins[f], acc)
    return acc


# speedup vs baseline: 3.0685x; 3.0685x over previous
import functools

import jax
import jax.numpy as jnp
from jax import lax
from jax.experimental import pallas as pl
from jax.experimental.pallas import tpu as pltpu
from jax.experimental.pallas import tpu_sc as plsc

BATCH = 16384
VOCABP1 = 100001
DIM = 64
FIELDS = 4
LANES = 16

_NC = 2
_NS = 16
_NW = _NC * _NS
_R = BATCH // _NW
_CG = DIM // LANES

_TBLK = 4096
_THALF = _TBLK // 2
_TGRID = (VOCABP1 + _TBLK - 1) // _TBLK

_mesh = plsc.VectorSubcoreMesh(core_axis_name="c", subcore_axis_name="s")
_params = pltpu.CompilerParams(use_tc_tiling_on_sc=False)


_LINROWS = _THALF * _TGRID


def _transpose_body(tt_ref, out_ref):
    x = tt_ref[...]
    eye = jnp.eye(DIM, dtype=jnp.float32)
    y = jax.lax.dot_general(x, eye, (((0,), (0,)), ((), ())),
                            preferred_element_type=jnp.float32)
    out_ref[...] = jnp.concatenate([y[:_THALF, :], y[_THALF:, :]], axis=1)


_transpose_flat = pl.pallas_call(
    _transpose_body,
    grid=(_TGRID,),
    in_specs=[pl.BlockSpec((DIM, _TBLK), lambda j: (0, j))],
    out_specs=pl.BlockSpec((_THALF, 128), lambda j: (j, 0)),
    out_shape=jax.ShapeDtypeStruct((_LINROWS, 128), jnp.float32),
)


def _permute_indices(idx_v):
    def ibody(i, carry):
        v = idx_v[pl.ds(i * LANES, LANES)]
        t = v & (_TBLK - 1)
        two_t = t + t
        idx_v[pl.ds(i * LANES, LANES)] = (v - t) + jnp.where(
            t < _THALF, two_t, two_t - (_TBLK - 1))
        return carry

    lax.fori_loop(0, _R // LANES, ibody, 0)


@functools.partial(
    pl.kernel,
    out_type=jax.ShapeDtypeStruct((BATCH, DIM), jnp.float32),
    mesh=_mesh,
    scratch_types=[
        pltpu.VMEM((_R,), jnp.int32),
        pltpu.VMEM((_R, DIM), jnp.float32),
        pltpu.SemaphoreType.DMA,
    ],
    compiler_params=_params,
)
def _gather_first(lab, tab, out, idx_v, gbuf, sem):
    wid = lax.axis_index("s") * _NC + lax.axis_index("c")
    base = wid * _R
    pltpu.sync_copy(lab.at[pl.ds(base, _R)], idx_v)
    _permute_indices(idx_v)
    pltpu.async_copy(tab.at[idx_v], gbuf, sem).wait()
    pltpu.sync_copy(gbuf, out.at[pl.ds(base, _R)])


@functools.partial(
    pl.kernel,
    out_type=jax.ShapeDtypeStruct((BATCH, DIM), jnp.float32),
    mesh=_mesh,
    scratch_types=[
        pltpu.VMEM((_R,), jnp.int32),
        pltpu.VMEM((_R, DIM), jnp.float32),
        pltpu.VMEM((_R, DIM), jnp.float32),
        pltpu.SemaphoreType.DMA,
        pltpu.SemaphoreType.DMA,
    ],
    compiler_params=_params,
)
def _gather_acc(lab, tab, acc, out, idx_v, gbuf, abuf, gsem, asem):
    wid = lax.axis_index("s") * _NC + lax.axis_index("c")
    base = wid * _R
    pltpu.sync_copy(lab.at[pl.ds(base, _R)], idx_v)
    _permute_indices(idx_v)
    gd = pltpu.async_copy(tab.at[idx_v], gbuf, gsem)
    ad = pltpu.async_copy(acc.at[pl.ds(base, _R)], abuf, asem)
    gd.wait()
    ad.wait()

    def body(r, carry):
        for cg in range(_CG):
            sl = pl.ds(cg * LANES, LANES)
            gbuf[r, sl] = gbuf[r, sl] + abuf[r, sl]
        return carry

    lax.fori_loop(0, _R, body, 0)
    pltpu.sync_copy(gbuf, out.at[pl.ds(base, _R)])


def kernel(labels_f0, labels_f1, labels_f2, labels_f3,
           table_f0, table_f1, table_f2, table_f3):
    labels = [labels_f0, labels_f1, labels_f2, labels_f3]
    tables = [table_f0, table_f1, table_f2, table_f3]
    lins = [_transpose_flat(t.T).reshape(2 * _LINROWS, DIM) for t in tables]
    acc = _gather_first(labels[0], lins[0])
    for f in range(1, FIELDS):
        acc = _gather_acc(labels[f], lins[f], acc)
    return acc


# speedup vs baseline: 3.5915x; 1.1704x over previous
import functools

import jax
import jax.numpy as jnp
from jax import lax
from jax.experimental import pallas as pl
from jax.experimental.pallas import tpu as pltpu
from jax.experimental.pallas import tpu_sc as plsc

BATCH = 16384
VOCABP1 = 100001
DIM = 64
FIELDS = 4
LANES = 16

_NC = 2
_NS = 16
_NW = _NC * _NS
_R = BATCH // _NW
_CG = DIM // LANES

_TBLK = 16384
_THALF = _TBLK // 2
_TGRID = (VOCABP1 + _TBLK - 1) // _TBLK

_mesh = plsc.VectorSubcoreMesh(core_axis_name="c", subcore_axis_name="s")
_params = pltpu.CompilerParams(use_tc_tiling_on_sc=False)


_LINROWS = _THALF * _TGRID


def _transpose_body(tt_ref, out_ref):
    x = tt_ref[...]
    eye = jnp.eye(DIM, dtype=jnp.float32)
    y = jax.lax.dot_general(x, eye, (((0,), (0,)), ((), ())),
                            preferred_element_type=jnp.float32)
    out_ref[...] = jnp.concatenate([y[:_THALF, :], y[_THALF:, :]], axis=1)


_transpose_flat = pl.pallas_call(
    _transpose_body,
    grid=(_TGRID,),
    in_specs=[pl.BlockSpec((DIM, _TBLK), lambda j: (0, j))],
    out_specs=pl.BlockSpec((_THALF, 128), lambda j: (j, 0)),
    out_shape=jax.ShapeDtypeStruct((_LINROWS, 128), jnp.float32),
)


def _permute_indices(idx_v):
    def ibody(i, carry):
        v = idx_v[pl.ds(i * LANES, LANES)]
        t = v & (_TBLK - 1)
        two_t = t + t
        idx_v[pl.ds(i * LANES, LANES)] = (v - t) + jnp.where(
            t < _THALF, two_t, two_t - (_TBLK - 1))
        return carry

    lax.fori_loop(0, _R // LANES, ibody, 0)


@functools.partial(
    pl.kernel,
    out_type=jax.ShapeDtypeStruct((BATCH, DIM), jnp.float32),
    mesh=_mesh,
    scratch_types=[
        pltpu.VMEM((_R,), jnp.int32),
        pltpu.VMEM((_R, DIM), jnp.float32),
        pltpu.SemaphoreType.DMA,
    ],
    compiler_params=_params,
)
def _gather_first(lab, tab, out, idx_v, gbuf, sem):
    wid = lax.axis_index("s") * _NC + lax.axis_index("c")
    base = wid * _R
    pltpu.sync_copy(lab.at[pl.ds(base, _R)], idx_v)
    _permute_indices(idx_v)
    pltpu.async_copy(tab.at[idx_v], gbuf, sem).wait()
    pltpu.sync_copy(gbuf, out.at[pl.ds(base, _R)])


@functools.partial(
    pl.kernel,
    out_type=jax.ShapeDtypeStruct((BATCH, DIM), jnp.float32),
    mesh=_mesh,
    scratch_types=[
        pltpu.VMEM((_R,), jnp.int32),
        pltpu.VMEM((_R, DIM), jnp.float32),
        pltpu.VMEM((_R, DIM), jnp.float32),
        pltpu.SemaphoreType.DMA,
        pltpu.SemaphoreType.DMA,
    ],
    compiler_params=_params,
)
def _gather_acc(lab, tab, acc, out, idx_v, gbuf, abuf, gsem, asem):
    wid = lax.axis_index("s") * _NC + lax.axis_index("c")
    base = wid * _R
    pltpu.sync_copy(lab.at[pl.ds(base, _R)], idx_v)
    _permute_indices(idx_v)
    gd = pltpu.async_copy(tab.at[idx_v], gbuf, gsem)
    ad = pltpu.async_copy(acc.at[pl.ds(base, _R)], abuf, asem)
    gd.wait()
    ad.wait()

    def body(r, carry):
        for cg in range(_CG):
            sl = pl.ds(cg * LANES, LANES)
            gbuf[r, sl] = gbuf[r, sl] + abuf[r, sl]
        return carry

    lax.fori_loop(0, _R, body, 0)
    pltpu.sync_copy(gbuf, out.at[pl.ds(base, _R)])


def kernel(labels_f0, labels_f1, labels_f2, labels_f3,
           table_f0, table_f1, table_f2, table_f3):
    labels = [labels_f0, labels_f1, labels_f2, labels_f3]
    tables = [table_f0, table_f1, table_f2, table_f3]
    lins = [_transpose_flat(t.T).reshape(2 * _LINROWS, DIM) for t in tables]
    acc = _gather_first(labels[0], lins[0])
    for f in range(1, FIELDS):
        acc = _gather_acc(labels[f], lins[f], acc)
    return acc
